# baseline (speedup 1.0000x reference)
import functools
import jax
import jax.numpy as jnp
from jax import lax
from jax.experimental import pallas as pl
from jax.experimental.pallas import tpu as pltpu
from jax.experimental.pallas import tpu_sc as plsc

N = 50000
E = 800000
HID = 64
OUT = 6
G = 256

NC = 2
NS = 16
LANES = 16

NPAD = 50176
CPT = NPAD // NS
UNITS = NPAD // 1024
ROWS = 6656
EPAD = ROWS * 128
KS = 8
RPT_HALF = ROWS // (NC * NS)
RPT_FULL = ROWS // NS

_mesh = plsc.VectorSubcoreMesh(
    core_axis_name="c", subcore_axis_name="s", num_cores=NC, num_subcores=NS)

_f32 = jnp.float32


def _zero_fill(ref, n):
  zeros = jnp.zeros((LANES,), _f32)
  def body(i, carry):
    ref[pl.ds(i * LANES, LANES)] = zeros
    return carry
  lax.fori_loop(0, n // LANES, body, 0, unroll=4)


def _unit_loop(sid, body):
  n = (UNITS - sid + NS - 1) // NS
  def b(k, carry):
    body(sid + k * NS)
    return carry
  lax.fori_loop(0, n, b, 0)


def _zero_fill2(ref, rows, cols):
  zeros = jnp.zeros((LANES,), _f32)
  per_row = cols // LANES
  def body(i, carry):
    r = i // per_row
    q = i - r * per_row
    ref[r, pl.ds(q * LANES, LANES)] = zeros
    return carry
  lax.fori_loop(0, rows * per_row, body, 0, unroll=4)


@functools.partial(
    pl.kernel,
    out_type=jax.ShapeDtypeStruct((NC * NPAD,), _f32),
    mesh=_mesh,
    scratch_types=[
        pltpu.VMEM((KS, 128), jnp.int32),
        pltpu.VMEM((128,), _f32),
        pltpu.VMEM((1024,), _f32),
        pltpu.VMEM_SHARED((NPAD,), _f32),
        pltpu.SemaphoreType.DMA,
        pltpu.SemaphoreType.DMA,
    ],
)
def _k1_deg(dst_hbm, out_hbm, dstbuf, onesbuf, obuf, acc, sem_in, sem_sc):
  cid = lax.axis_index("c")
  sid = lax.axis_index("s")
  wid = cid * NS + sid

  ones = jnp.ones((LANES,), _f32)
  for q in range(128 // LANES):
    onesbuf[pl.ds(q * LANES, LANES)] = ones

  _zero_fill(obuf, 1024)
  _unit_loop(sid, lambda u: pltpu.sync_copy(obuf, acc.at[pl.ds(u * 1024, 1024)]))
  plsc.subcore_barrier()

  base = wid * RPT_HALF
  def stage(st, carry):
    cp = pltpu.async_copy(
        dst_hbm.at[pl.ds(base + st * KS, KS)], dstbuf, sem_in)
    cp.wait()
    cps = [pltpu.async_copy(onesbuf, acc.at[dstbuf.at[j]], sem_sc, add=True)
           for j in range(KS)]
    for cp2 in cps:
      cp2.wait()
    return carry
  lax.fori_loop(0, RPT_HALF // KS, stage, 0)

  plsc.subcore_barrier()

  def rb(u):
    pltpu.sync_copy(acc.at[pl.ds(u * 1024, 1024)], obuf)
    pltpu.sync_copy(obuf, out_hbm.at[pl.ds(cid * NPAD + u * 1024, 1024)])
  _unit_loop(sid, rb)


@functools.partial(
    pl.kernel,
    out_type=jax.ShapeDtypeStruct((NC * NPAD,), _f32),
    mesh=_mesh,
    scratch_types=[
        pltpu.VMEM((KS, 128), jnp.int32),
        pltpu.VMEM((KS, 128), jnp.int32),
        pltpu.VMEM((KS, 128), _f32),
        pltpu.VMEM((1024,), _f32),
        pltpu.VMEM_SHARED((NPAD,), _f32),
        pltpu.SemaphoreType.DMA,
        pltpu.SemaphoreType.DMA,
        pltpu.SemaphoreType.DMA,
    ],
)
def _k3_tpre(src_hbm, dst_hbm, u_hbm, out_hbm,
             srcbuf, dstbuf, valbuf, obuf, acc, sem_in, sem_g, sem_sc):
  cid = lax.axis_index("c")
  sid = lax.axis_index("s")
  wid = cid * NS + sid

  _zero_fill(obuf, 1024)
  _unit_loop(sid, lambda u: pltpu.sync_copy(obuf, acc.at[pl.ds(u * 1024, 1024)]))
  plsc.subcore_barrier()

  base = wid * RPT_HALF
  def stage(st, carry):
    r0 = base + st * KS
    cpa = pltpu.async_copy(src_hbm.at[pl.ds(r0, KS)], srcbuf, sem_in)
    cpb = pltpu.async_copy(dst_hbm.at[pl.ds(r0, KS)], dstbuf, sem_in)
    cpa.wait()
    cpb.wait()
    gets = [pltpu.async_copy(u_hbm.at[srcbuf.at[j]], valbuf.at[j], sem_g)
            for j in range(KS)]
    for cp in gets:
      cp.wait()
    puts = [pltpu.async_copy(valbuf.at[j], acc.at[dstbuf.at[j]], sem_sc,
                             add=True)
            for j in range(KS)]
    for cp in puts:
      cp.wait()
    return carry
  lax.fori_loop(0, RPT_HALF // KS, stage, 0)

  plsc.subcore_barrier()

  def rb(u):
    pltpu.sync_copy(acc.at[pl.ds(u * 1024, 1024)], obuf)
    pltpu.sync_copy(obuf, out_hbm.at[pl.ds(cid * NPAD + u * 1024, 1024)])
  _unit_loop(sid, rb)


ZR = 392
FQ = 16


KS5 = 16
NST5 = RPT_FULL // KS5


@functools.partial(
    pl.kernel,
    out_type=jax.ShapeDtypeStruct((NC, NPAD, FQ), _f32),
    mesh=_mesh,
    scratch_types=[
        pltpu.VMEM((KS5, 128), jnp.int32),
        pltpu.VMEM((KS5, 128), jnp.int32),
        pltpu.VMEM((KS5, 128, FQ), _f32),
        pltpu.VMEM((ZR, FQ), _f32),
        pltpu.VMEM_SHARED((NPAD, FQ), _f32),
        pltpu.SemaphoreType.DMA,
        pltpu.SemaphoreType.DMA,
        pltpu.SemaphoreType.DMA,
    ],
    compiler_params=pltpu.CompilerParams(use_tc_tiling_on_sc=False),
)
def _k5_acc(src_hbm, dst_hbm, g_hbm, out_hbm,
            srcbuf, dstbuf, rowbuf, obuf, acc, sem_in, sem_g, sem_sc):
  cid = lax.axis_index("c")
  sid = lax.axis_index("s")
  off = (cid * NPAD).astype(jnp.int32)

  _zero_fill2(obuf, ZR, FQ)
  for i in range(CPT // ZR):
    pltpu.sync_copy(obuf, acc.at[pl.ds(sid * CPT + i * ZR, ZR)])
  plsc.subcore_barrier()

  base = sid * RPT_FULL
  offv = jnp.full((LANES,), 0, jnp.int32) + off

  def stage(st, carry):
    r0 = base + st * KS5
    cpa = pltpu.async_copy(src_hbm.at[pl.ds(r0, KS5)], srcbuf, sem_in)
    cpb = pltpu.async_copy(dst_hbm.at[pl.ds(r0, KS5)], dstbuf, sem_in)
    cpa.wait()
    cpb.wait()
    def addoff(i, c):
      r = i // 8
      q = i - r * 8
      srcbuf[r, pl.ds(q * LANES, LANES)] = (
          srcbuf[r, pl.ds(q * LANES, LANES)] + offv)
      return c
    lax.fori_loop(0, KS5 * 8, addoff, 0, unroll=8)
    gets = [pltpu.async_copy(g_hbm.at[srcbuf.at[j]], rowbuf.at[j], sem_g)
            for j in range(KS5)]
    for cp in gets:
      cp.wait()
    puts = [pltpu.async_copy(rowbuf.at[j], acc.at[dstbuf.at[j]], sem_sc,
                             add=True)
            for j in range(KS5)]
    for cp in puts:
      cp.wait()
    return carry
  lax.fori_loop(0, NST5, stage, 0)

  plsc.subcore_barrier()
  for i in range(CPT // ZR):
    pltpu.sync_copy(acc.at[pl.ds(sid * CPT + i * ZR, ZR)], obuf)
    pltpu.sync_copy(obuf, out_hbm.at[cid, pl.ds(sid * CPT + i * ZR, ZR)])


def _k2_body(degs_ref, x_ref, dinv_ref, u_ref):
  deg = jnp.maximum(degs_ref[0] + degs_ref[1], 1.0)
  dinv = lax.rsqrt(deg)
  dinv_ref[...] = dinv
  u_ref[...] = dinv * x_ref[...]


def _k2_call(degs2, x2):
  return pl.pallas_call(
      _k2_body,
      out_shape=(jax.ShapeDtypeStruct((392, 128), _f32),
                 jax.ShapeDtypeStruct((392, 128), _f32)),
  )(degs2, x2)


BROWS = 1024
NBLK = NPAD // BROWS


def _k4_body(tp0_ref, tp1_ref, dinv_ref, w1_ref, b1_ref, w2_ref,
             gq_ref):
  dinv = dinv_ref[...]
  t = dinv * (tp0_ref[...] + tp1_ref[...])
  h1 = jnp.maximum(t * w1_ref[...] + b1_ref[...], 0.0)
  hm = jnp.dot(h1, w2_ref[...], preferred_element_type=_f32)
  g = dinv * hm
  for q in range(4):
    gq_ref[q] = g[:, q * FQ:(q + 1) * FQ]


def _k4_call(tp0, tp1, dinv, w1, b1, w2):
  col = pl.BlockSpec((BROWS, 1), lambda i: (i, 0))
  full = lambda r, c: pl.BlockSpec((r, c), lambda i: (0, 0))
  return pl.pallas_call(
      _k4_body,
      grid=(NBLK,),
      in_specs=[col, col, col, full(1, HID), full(1, HID),
                full(HID, HID)],
      out_specs=pl.BlockSpec((4, BROWS, FQ), lambda i: (0, i, 0)),
      out_shape=jax.ShapeDtypeStruct((4, NPAD, FQ), _f32),
  )(tp0, tp1, dinv, w1, b1, w2)


B6 = 3136
NB6 = NPAD // B6


def _k6_body(a01_ref, a23_ref, dinv_ref, bat_ref, b2_ref,
             wl1_ref, bl1_ref, wl2_ref, bl2_ref, out_ref, pacc, cacc):
  i = pl.program_id(0)

  @pl.when(i == 0)
  def _():
    pacc[...] = jnp.zeros_like(pacc)
    cacc[...] = jnp.zeros_like(cacc)

  ag = jnp.concatenate(
      [a01_ref[0], a01_ref[1], a23_ref[0], a23_ref[1]], axis=1)
  h2 = jnp.maximum(dinv_ref[...] * ag + b2_ref[...], 0.0)
  gids = lax.broadcasted_iota(jnp.int32, (B6, G), 1)
  ind = (bat_ref[...] == gids).astype(_f32)
  dn = (((0,), (0,)), ((), ()))
  pacc[...] += lax.dot_general(ind, h2, dn, preferred_element_type=_f32)
  ones = jnp.ones((B6, 1), _f32)
  cacc[...] += lax.dot_general(ind, ones, dn, preferred_element_type=_f32)

  @pl.when(i == NB6 - 1)
  def _():
    pooled = pacc[...] / jnp.maximum(cacc[...], 1.0)
    hp = jnp.maximum(
        jnp.dot(pooled, wl1_ref[...], preferred_element_type=_f32)
        + bl1_ref[...], 0.0)
    out_ref[...] = (jnp.dot(hp, wl2_ref[...], preferred_element_type=_f32)
                    + bl2_ref[...])


def _k6_call(a01, a23, dinv, bat, b2, wl1, bl1, wl2p, bl2p):
  blk2 = pl.BlockSpec((2, B6, FQ), lambda i: (0, i, 0))
  blk = lambda c: pl.BlockSpec((B6, c), lambda i: (i, 0))
  full = lambda r, c: pl.BlockSpec((r, c), lambda i: (0, 0))
  return pl.pallas_call(
      _k6_body,
      grid=(NB6,),
      in_specs=[blk2, blk2, blk(1), blk(1),
                full(1, HID), full(HID, HID), full(1, HID),
                full(HID, 128), full(1, 128)],
      out_specs=full(G, 128),
      out_shape=jax.ShapeDtypeStruct((G, 128), _f32),
      scratch_shapes=[pltpu.VMEM((G, HID), _f32), pltpu.VMEM((G, 1), _f32)],
  )(a01, a23, dinv, bat, b2, wl1, bl1, wl2p, bl2p)


def kernel(x, edge_index, batch, W1, b1, W2, b2, Wl1, bl1, Wl2, bl2):
  src = edge_index[0].astype(jnp.int32)
  dst = edge_index[1].astype(jnp.int32)
  bat = batch.astype(jnp.int32)

  loop = jnp.arange(N, dtype=jnp.int32)
  npad_e = EPAD - E - N
  ar = jnp.arange(npad_e, dtype=jnp.int32)
  src_p = jnp.concatenate([src, loop, ar % N]).reshape(ROWS, 128)
  dst_p = jnp.concatenate(
      [dst, loop, N + (ar % (NPAD - N))]).reshape(ROWS, 128)

  x_p = jnp.concatenate([x[:, 0], jnp.zeros((NPAD - N,), _f32)])
  bat_p = jnp.concatenate(
      [bat, jnp.full((NPAD - N,), G, jnp.int32)]).reshape(NPAD, 1)

  degs = _k1_deg(dst_p)
  dinv2, u2 = _k2_call(degs.reshape(2, 392, 128), x_p.reshape(392, 128))
  dinv_f = dinv2.reshape(NPAD)
  u_f = u2.reshape(NPAD)

  tps = _k3_tpre(src_p, dst_p, u_f)

  dinv_c = dinv_f.reshape(NPAD, 1)
  gq = _k4_call(tps[:NPAD].reshape(NPAD, 1), tps[NPAD:].reshape(NPAD, 1),
                dinv_c, W1, b1.reshape(1, HID), W2)

  accs01 = _k5_acc(src_p, dst_p, gq[:2].reshape(2 * NPAD, FQ))
  accs23 = _k5_acc(src_p, dst_p, gq[2:].reshape(2 * NPAD, FQ))

  wl2p = jnp.concatenate([Wl2, jnp.zeros((HID, 128 - OUT), _f32)], axis=1)
  bl2p = jnp.concatenate([bl2, jnp.zeros((128 - OUT,), _f32)]).reshape(1, 128)
  outp = _k6_call(accs01, accs23, dinv_c, bat_p,
                  b2.reshape(1, HID), Wl1, bl1.reshape(1, HID), wl2p, bl2p)
  return outp[:, :OUT]


# speedup vs baseline: 1.0573x; 1.0573x over previous
import functools
import jax
import jax.numpy as jnp
from jax import lax
from jax.experimental import pallas as pl
from jax.experimental.pallas import tpu as pltpu
from jax.experimental.pallas import tpu_sc as plsc

N = 50000
E = 800000
HID = 64
OUT = 6
G = 256

NC = 2
NS = 16
LANES = 16

NPAD = 50176
CPT = NPAD // NS
UNITS = NPAD // 1024
ROWS = 6656
EPAD = ROWS * 128
KS = 8
RPT_HALF = ROWS // (NC * NS)
RPT_FULL = ROWS // NS

_mesh = plsc.VectorSubcoreMesh(
    core_axis_name="c", subcore_axis_name="s", num_cores=NC, num_subcores=NS)

_f32 = jnp.float32


def _zero_fill(ref, n):
  zeros = jnp.zeros((LANES,), _f32)
  def body(i, carry):
    ref[pl.ds(i * LANES, LANES)] = zeros
    return carry
  lax.fori_loop(0, n // LANES, body, 0, unroll=4)


def _unit_loop(sid, body):
  n = (UNITS - sid + NS - 1) // NS
  def b(k, carry):
    body(sid + k * NS)
    return carry
  lax.fori_loop(0, n, b, 0)


def _zero_fill2(ref, rows, cols):
  zeros = jnp.zeros((LANES,), _f32)
  per_row = cols // LANES
  def body(i, carry):
    r = i // per_row
    q = i - r * per_row
    ref[r, pl.ds(q * LANES, LANES)] = zeros
    return carry
  lax.fori_loop(0, rows * per_row, body, 0, unroll=4)


@functools.partial(
    pl.kernel,
    out_type=jax.ShapeDtypeStruct((NC * NPAD,), _f32),
    mesh=_mesh,
    scratch_types=[
        pltpu.VMEM((KS, 128), jnp.int32),
        pltpu.VMEM((128,), _f32),
        pltpu.VMEM((1024,), _f32),
        pltpu.VMEM_SHARED((NPAD,), _f32),
        pltpu.SemaphoreType.DMA,
        pltpu.SemaphoreType.DMA,
    ],
)
def _k1_deg(dst_hbm, out_hbm, dstbuf, onesbuf, obuf, acc, sem_in, sem_sc):
  cid = lax.axis_index("c")
  sid = lax.axis_index("s")
  wid = cid * NS + sid

  ones = jnp.ones((LANES,), _f32)
  for q in range(128 // LANES):
    onesbuf[pl.ds(q * LANES, LANES)] = ones

  _zero_fill(obuf, 1024)
  _unit_loop(sid, lambda u: pltpu.sync_copy(obuf, acc.at[pl.ds(u * 1024, 1024)]))
  plsc.subcore_barrier()

  base = wid * RPT_HALF
  def stage(st, carry):
    cp = pltpu.async_copy(
        dst_hbm.at[pl.ds(base + st * KS, KS)], dstbuf, sem_in)
    cp.wait()
    cps = [pltpu.async_copy(onesbuf, acc.at[dstbuf.at[j]], sem_sc, add=True)
           for j in range(KS)]
    for cp2 in cps:
      cp2.wait()
    return carry
  lax.fori_loop(0, RPT_HALF // KS, stage, 0)

  plsc.subcore_barrier()

  def rb(u):
    pltpu.sync_copy(acc.at[pl.ds(u * 1024, 1024)], obuf)
    pltpu.sync_copy(obuf, out_hbm.at[pl.ds(cid * NPAD + u * 1024, 1024)])
  _unit_loop(sid, rb)


@functools.partial(
    pl.kernel,
    out_type=jax.ShapeDtypeStruct((NC * NPAD,), _f32),
    mesh=_mesh,
    scratch_types=[
        pltpu.VMEM((KS, 128), jnp.int32),
        pltpu.VMEM((KS, 128), jnp.int32),
        pltpu.VMEM((KS, 128), _f32),
        pltpu.VMEM((1024,), _f32),
        pltpu.VMEM_SHARED((NPAD,), _f32),
        pltpu.SemaphoreType.DMA,
        pltpu.SemaphoreType.DMA,
        pltpu.SemaphoreType.DMA,
    ],
)
def _k3_tpre(src_hbm, dst_hbm, u_hbm, out_hbm,
             srcbuf, dstbuf, valbuf, obuf, acc, sem_in, sem_g, sem_sc):
  cid = lax.axis_index("c")
  sid = lax.axis_index("s")
  wid = cid * NS + sid

  _zero_fill(obuf, 1024)
  _unit_loop(sid, lambda u: pltpu.sync_copy(obuf, acc.at[pl.ds(u * 1024, 1024)]))
  plsc.subcore_barrier()

  base = wid * RPT_HALF
  def stage(st, carry):
    r0 = base + st * KS
    cpa = pltpu.async_copy(src_hbm.at[pl.ds(r0, KS)], srcbuf, sem_in)
    cpb = pltpu.async_copy(dst_hbm.at[pl.ds(r0, KS)], dstbuf, sem_in)
    cpa.wait()
    cpb.wait()
    H = KS // 2
    gets = [pltpu.async_copy(u_hbm.at[srcbuf.at[j]], valbuf.at[j], sem_g)
            for j in range(KS)]
    for cp in gets[:H]:
      cp.wait()
    puts_a = [pltpu.async_copy(valbuf.at[j], acc.at[dstbuf.at[j]], sem_sc,
                               add=True)
              for j in range(H)]
    for cp in gets[H:]:
      cp.wait()
    puts_b = [pltpu.async_copy(valbuf.at[j], acc.at[dstbuf.at[j]], sem_sc,
                               add=True)
              for j in range(H, KS)]
    for cp in puts_a + puts_b:
      cp.wait()
    return carry
  lax.fori_loop(0, RPT_HALF // KS, stage, 0)

  plsc.subcore_barrier()

  def rb(u):
    pltpu.sync_copy(acc.at[pl.ds(u * 1024, 1024)], obuf)
    pltpu.sync_copy(obuf, out_hbm.at[pl.ds(cid * NPAD + u * 1024, 1024)])
  _unit_loop(sid, rb)


ZR = 392
FQ = 16


KS5 = 16
NST5 = RPT_FULL // KS5


@functools.partial(
    pl.kernel,
    out_type=jax.ShapeDtypeStruct((NC, NPAD, FQ), _f32),
    mesh=_mesh,
    scratch_types=[
        pltpu.VMEM((KS5, 128), jnp.int32),
        pltpu.VMEM((KS5, 128), jnp.int32),
        pltpu.VMEM((KS5, 128, FQ), _f32),
        pltpu.VMEM((ZR, FQ), _f32),
        pltpu.VMEM_SHARED((NPAD, FQ), _f32),
        pltpu.SemaphoreType.DMA,
        pltpu.SemaphoreType.DMA,
        pltpu.SemaphoreType.DMA,
    ],
    compiler_params=pltpu.CompilerParams(use_tc_tiling_on_sc=False),
)
def _k5_acc(src_hbm, dst_hbm, g_hbm, out_hbm,
            srcbuf, dstbuf, rowbuf, obuf, acc, sem_in, sem_g, sem_sc):
  cid = lax.axis_index("c")
  sid = lax.axis_index("s")
  off = (cid * NPAD).astype(jnp.int32)

  _zero_fill2(obuf, ZR, FQ)
  for i in range(CPT // ZR):
    pltpu.sync_copy(obuf, acc.at[pl.ds(sid * CPT + i * ZR, ZR)])
  plsc.subcore_barrier()

  base = sid * RPT_FULL
  offv = jnp.full((LANES,), 0, jnp.int32) + off

  def stage(st, carry):
    r0 = base + st * KS5
    cpa = pltpu.async_copy(src_hbm.at[pl.ds(r0, KS5)], srcbuf, sem_in)
    cpb = pltpu.async_copy(dst_hbm.at[pl.ds(r0, KS5)], dstbuf, sem_in)
    cpa.wait()
    cpb.wait()
    def addoff(i, c):
      r = i // 8
      q = i - r * 8
      srcbuf[r, pl.ds(q * LANES, LANES)] = (
          srcbuf[r, pl.ds(q * LANES, LANES)] + offv)
      return c
    lax.fori_loop(0, KS5 * 8, addoff, 0, unroll=8)
    H = KS5 // 2
    gets = [pltpu.async_copy(g_hbm.at[srcbuf.at[j]], rowbuf.at[j], sem_g)
            for j in range(KS5)]
    for cp in gets[:H]:
      cp.wait()
    puts_a = [pltpu.async_copy(rowbuf.at[j], acc.at[dstbuf.at[j]], sem_sc,
                               add=True)
              for j in range(H)]
    for cp in gets[H:]:
      cp.wait()
    puts_b = [pltpu.async_copy(rowbuf.at[j], acc.at[dstbuf.at[j]], sem_sc,
                               add=True)
              for j in range(H, KS5)]
    for cp in puts_a + puts_b:
      cp.wait()
    return carry
  lax.fori_loop(0, NST5, stage, 0)

  plsc.subcore_barrier()
  for i in range(CPT // ZR):
    pltpu.sync_copy(acc.at[pl.ds(sid * CPT + i * ZR, ZR)], obuf)
    pltpu.sync_copy(obuf, out_hbm.at[cid, pl.ds(sid * CPT + i * ZR, ZR)])


def _k2_body(degs_ref, x_ref, dinv_ref, u_ref):
  deg = jnp.maximum(degs_ref[0] + degs_ref[1], 1.0)
  dinv = lax.rsqrt(deg)
  dinv_ref[...] = dinv
  u_ref[...] = dinv * x_ref[...]


def _k2_call(degs2, x2):
  return pl.pallas_call(
      _k2_body,
      out_shape=(jax.ShapeDtypeStruct((392, 128), _f32),
                 jax.ShapeDtypeStruct((392, 128), _f32)),
  )(degs2, x2)


BROWS = 3136
NBLK = NPAD // BROWS


def _k4_body(tp0_ref, tp1_ref, dinv_ref, w1_ref, b1_ref, w2_ref,
             gq_ref):
  dinv = dinv_ref[...]
  t = dinv * (tp0_ref[...] + tp1_ref[...])
  h1 = jnp.maximum(t * w1_ref[...] + b1_ref[...], 0.0)
  hm = jnp.dot(h1, w2_ref[...], preferred_element_type=_f32)
  g = dinv * hm
  for q in range(4):
    gq_ref[q] = g[:, q * FQ:(q + 1) * FQ]


def _k4_call(tp0, tp1, dinv, w1, b1, w2):
  col = pl.BlockSpec((BROWS, 1), lambda i: (i, 0))
  full = lambda r, c: pl.BlockSpec((r, c), lambda i: (0, 0))
  return pl.pallas_call(
      _k4_body,
      grid=(NBLK,),
      in_specs=[col, col, col, full(1, HID), full(1, HID),
                full(HID, HID)],
      out_specs=pl.BlockSpec((4, BROWS, FQ), lambda i: (0, i, 0)),
      out_shape=jax.ShapeDtypeStruct((4, NPAD, FQ), _f32),
  )(tp0, tp1, dinv, w1, b1, w2)


B6 = 3136
NB6 = NPAD // B6


def _k6_body(a01_ref, a23_ref, dinv_ref, bat_ref, b2_ref,
             wl1_ref, bl1_ref, wl2_ref, bl2_ref, out_ref, pacc, cacc):
  i = pl.program_id(0)

  @pl.when(i == 0)
  def _():
    pacc[...] = jnp.zeros_like(pacc)
    cacc[...] = jnp.zeros_like(cacc)

  ag = jnp.concatenate(
      [a01_ref[0], a01_ref[1], a23_ref[0], a23_ref[1]], axis=1)
  h2 = jnp.maximum(dinv_ref[...] * ag + b2_ref[...], 0.0)
  gids = lax.broadcasted_iota(jnp.int32, (B6, G), 1)
  ind = (bat_ref[...] == gids).astype(_f32)
  dn = (((0,), (0,)), ((), ()))
  pacc[...] += lax.dot_general(ind, h2, dn, preferred_element_type=_f32)
  ones = jnp.ones((B6, 1), _f32)
  cacc[...] += lax.dot_general(ind, ones, dn, preferred_element_type=_f32)

  @pl.when(i == NB6 - 1)
  def _():
    pooled = pacc[...] / jnp.maximum(cacc[...], 1.0)
    hp = jnp.maximum(
        jnp.dot(pooled, wl1_ref[...], preferred_element_type=_f32)
        + bl1_ref[...], 0.0)
    out_ref[...] = (jnp.dot(hp, wl2_ref[...], preferred_element_type=_f32)
                    + bl2_ref[...])


def _k6_call(a01, a23, dinv, bat, b2, wl1, bl1, wl2p, bl2p):
  blk2 = pl.BlockSpec((2, B6, FQ), lambda i: (0, i, 0))
  blk = lambda c: pl.BlockSpec((B6, c), lambda i: (i, 0))
  full = lambda r, c: pl.BlockSpec((r, c), lambda i: (0, 0))
  return pl.pallas_call(
      _k6_body,
      grid=(NB6,),
      in_specs=[blk2, blk2, blk(1), blk(1),
                full(1, HID), full(HID, HID), full(1, HID),
                full(HID, 128), full(1, 128)],
      out_specs=full(G, 128),
      out_shape=jax.ShapeDtypeStruct((G, 128), _f32),
      scratch_shapes=[pltpu.VMEM((G, HID), _f32), pltpu.VMEM((G, 1), _f32)],
  )(a01, a23, dinv, bat, b2, wl1, bl1, wl2p, bl2p)


def kernel(x, edge_index, batch, W1, b1, W2, b2, Wl1, bl1, Wl2, bl2):
  src = edge_index[0].astype(jnp.int32)
  dst = edge_index[1].astype(jnp.int32)
  bat = batch.astype(jnp.int32)

  loop = jnp.arange(N, dtype=jnp.int32)
  npad_e = EPAD - E - N
  ar = jnp.arange(npad_e, dtype=jnp.int32)
  src_p = jnp.concatenate([src, loop, ar % N]).reshape(ROWS, 128)
  dst_p = jnp.concatenate(
      [dst, loop, N + (ar % (NPAD - N))]).reshape(ROWS, 128)

  x_p = jnp.concatenate([x[:, 0], jnp.zeros((NPAD - N,), _f32)])
  bat_p = jnp.concatenate(
      [bat, jnp.full((NPAD - N,), G, jnp.int32)]).reshape(NPAD, 1)

  degs = _k1_deg(dst_p)
  dinv2, u2 = _k2_call(degs.reshape(2, 392, 128), x_p.reshape(392, 128))
  dinv_f = dinv2.reshape(NPAD)
  u_f = u2.reshape(NPAD)

  tps = _k3_tpre(src_p, dst_p, u_f)

  dinv_c = dinv_f.reshape(NPAD, 1)
  gq = _k4_call(tps[:NPAD].reshape(NPAD, 1), tps[NPAD:].reshape(NPAD, 1),
                dinv_c, W1, b1.reshape(1, HID), W2)

  accs01 = _k5_acc(src_p, dst_p, gq[:2].reshape(2 * NPAD, FQ))
  accs23 = _k5_acc(src_p, dst_p, gq[2:].reshape(2 * NPAD, FQ))

  wl2p = jnp.concatenate([Wl2, jnp.zeros((HID, 128 - OUT), _f32)], axis=1)
  bl2p = jnp.concatenate([bl2, jnp.zeros((128 - OUT,), _f32)]).reshape(1, 128)
  outp = _k6_call(accs01, accs23, dinv_c, bat_p,
                  b2.reshape(1, HID), Wl1, bl1.reshape(1, HID), wl2p, bl2p)
  return outp[:, :OUT]


# speedup vs baseline: 1.0950x; 1.0356x over previous
import functools
import jax
import jax.numpy as jnp
from jax import lax
from jax.experimental import pallas as pl
from jax.experimental.pallas import tpu as pltpu
from jax.experimental.pallas import tpu_sc as plsc

N = 50000
E = 800000
HID = 64
OUT = 6
G = 256

NC = 2
NS = 16
LANES = 16

NPAD = 50176
CPT = NPAD // NS
UNITS = NPAD // 1024
ROWS = 6656
EPAD = ROWS * 128
KS = 16
RPT_HALF = ROWS // (NC * NS)
RPT_FULL = ROWS // NS

_mesh = plsc.VectorSubcoreMesh(
    core_axis_name="c", subcore_axis_name="s", num_cores=NC, num_subcores=NS)

_f32 = jnp.float32


def _zero_fill(ref, n):
  zeros = jnp.zeros((LANES,), _f32)
  def body(i, carry):
    ref[pl.ds(i * LANES, LANES)] = zeros
    return carry
  lax.fori_loop(0, n // LANES, body, 0, unroll=4)


def _unit_loop(sid, body):
  n = (UNITS - sid + NS - 1) // NS
  def b(k, carry):
    body(sid + k * NS)
    return carry
  lax.fori_loop(0, n, b, 0)


def _zero_fill2(ref, rows, cols):
  zeros = jnp.zeros((LANES,), _f32)
  per_row = cols // LANES
  def body(i, carry):
    r = i // per_row
    q = i - r * per_row
    ref[r, pl.ds(q * LANES, LANES)] = zeros
    return carry
  lax.fori_loop(0, rows * per_row, body, 0, unroll=4)


@functools.partial(
    pl.kernel,
    out_type=jax.ShapeDtypeStruct((NC * NPAD,), _f32),
    mesh=_mesh,
    scratch_types=[
        pltpu.VMEM((KS, 128), jnp.int32),
        pltpu.VMEM((128,), _f32),
        pltpu.VMEM((1024,), _f32),
        pltpu.VMEM_SHARED((NPAD,), _f32),
        pltpu.SemaphoreType.DMA,
        pltpu.SemaphoreType.DMA,
    ],
)
def _k1_deg(dst_hbm, out_hbm, dstbuf, onesbuf, obuf, acc, sem_in, sem_sc):
  cid = lax.axis_index("c")
  sid = lax.axis_index("s")
  wid = cid * NS + sid

  ones = jnp.ones((LANES,), _f32)
  for q in range(128 // LANES):
    onesbuf[pl.ds(q * LANES, LANES)] = ones

  _zero_fill(obuf, 1024)
  _unit_loop(sid, lambda u: pltpu.sync_copy(obuf, acc.at[pl.ds(u * 1024, 1024)]))
  plsc.subcore_barrier()

  base = wid * RPT_HALF
  def stage(st, carry):
    cp = pltpu.async_copy(
        dst_hbm.at[pl.ds(base + st * KS, KS)], dstbuf, sem_in)
    cp.wait()
    cps = [pltpu.async_copy(onesbuf, acc.at[dstbuf.at[j]], sem_sc, add=True)
           for j in range(KS)]
    for cp2 in cps:
      cp2.wait()
    return carry
  lax.fori_loop(0, RPT_HALF // KS, stage, 0)

  plsc.subcore_barrier()

  def rb(u):
    pltpu.sync_copy(acc.at[pl.ds(u * 1024, 1024)], obuf)
    pltpu.sync_copy(obuf, out_hbm.at[pl.ds(cid * NPAD + u * 1024, 1024)])
  _unit_loop(sid, rb)


@functools.partial(
    pl.kernel,
    out_type=jax.ShapeDtypeStruct((NC * NPAD,), _f32),
    mesh=_mesh,
    scratch_types=[
        pltpu.VMEM((KS, 128), jnp.int32),
        pltpu.VMEM((KS, 128), jnp.int32),
        pltpu.VMEM((KS, 128), _f32),
        pltpu.VMEM((1024,), _f32),
        pltpu.VMEM_SHARED((NPAD,), _f32),
        pltpu.SemaphoreType.DMA,
        pltpu.SemaphoreType.DMA,
        pltpu.SemaphoreType.DMA,
    ],
)
def _k3_tpre(src_hbm, dst_hbm, u_hbm, out_hbm,
             srcbuf, dstbuf, valbuf, obuf, acc, sem_in, sem_g, sem_sc):
  cid = lax.axis_index("c")
  sid = lax.axis_index("s")
  wid = cid * NS + sid

  _zero_fill(obuf, 1024)
  _unit_loop(sid, lambda u: pltpu.sync_copy(obuf, acc.at[pl.ds(u * 1024, 1024)]))
  plsc.subcore_barrier()

  base = wid * RPT_HALF
  def stage(st, carry):
    r0 = base + st * KS
    cpa = pltpu.async_copy(src_hbm.at[pl.ds(r0, KS)], srcbuf, sem_in)
    cpb = pltpu.async_copy(dst_hbm.at[pl.ds(r0, KS)], dstbuf, sem_in)
    cpa.wait()
    cpb.wait()
    H = KS // 2
    gets = [pltpu.async_copy(u_hbm.at[srcbuf.at[j]], valbuf.at[j], sem_g)
            for j in range(KS)]
    for cp in gets[:H]:
      cp.wait()
    puts_a = [pltpu.async_copy(valbuf.at[j], acc.at[dstbuf.at[j]], sem_sc,
                               add=True)
              for j in range(H)]
    for cp in gets[H:]:
      cp.wait()
    puts_b = [pltpu.async_copy(valbuf.at[j], acc.at[dstbuf.at[j]], sem_sc,
                               add=True)
              for j in range(H, KS)]
    for cp in puts_a + puts_b:
      cp.wait()
    return carry
  lax.fori_loop(0, RPT_HALF // KS, stage, 0)

  plsc.subcore_barrier()

  def rb(u):
    pltpu.sync_copy(acc.at[pl.ds(u * 1024, 1024)], obuf)
    pltpu.sync_copy(obuf, out_hbm.at[pl.ds(cid * NPAD + u * 1024, 1024)])
  _unit_loop(sid, rb)


ZR = 392
FQ = 16


KS5 = 16
NST5 = RPT_FULL // KS5


@functools.partial(
    pl.kernel,
    out_type=jax.ShapeDtypeStruct((NC, NPAD, FQ), _f32),
    mesh=_mesh,
    scratch_types=[
        pltpu.VMEM((KS5, 128), jnp.int32),
        pltpu.VMEM((KS5, 128), jnp.int32),
        pltpu.VMEM((KS5, 128, FQ), _f32),
        pltpu.VMEM((ZR, FQ), _f32),
        pltpu.VMEM_SHARED((NPAD, FQ), _f32),
        pltpu.SemaphoreType.DMA,
        pltpu.SemaphoreType.DMA,
        pltpu.SemaphoreType.DMA,
    ],
    compiler_params=pltpu.CompilerParams(use_tc_tiling_on_sc=False),
)
def _k5_acc(src_hbm, dst_hbm, g_hbm, out_hbm,
            srcbuf, dstbuf, rowbuf, obuf, acc, sem_in, sem_g, sem_sc):
  cid = lax.axis_index("c")
  sid = lax.axis_index("s")
  off = (cid * NPAD).astype(jnp.int32)

  _zero_fill2(obuf, ZR, FQ)
  for i in range(CPT // ZR):
    pltpu.sync_copy(obuf, acc.at[pl.ds(sid * CPT + i * ZR, ZR)])
  plsc.subcore_barrier()

  base = sid * RPT_FULL
  offv = jnp.full((LANES,), 0, jnp.int32) + off

  def stage(st, carry):
    r0 = base + st * KS5
    cpa = pltpu.async_copy(src_hbm.at[pl.ds(r0, KS5)], srcbuf, sem_in)
    cpb = pltpu.async_copy(dst_hbm.at[pl.ds(r0, KS5)], dstbuf, sem_in)
    cpa.wait()
    cpb.wait()
    def addoff(i, c):
      r = i // 8
      q = i - r * 8
      srcbuf[r, pl.ds(q * LANES, LANES)] = (
          srcbuf[r, pl.ds(q * LANES, LANES)] + offv)
      return c
    lax.fori_loop(0, KS5 * 8, addoff, 0, unroll=8)
    H = KS5 // 2
    gets = [pltpu.async_copy(g_hbm.at[srcbuf.at[j]], rowbuf.at[j], sem_g)
            for j in range(KS5)]
    for cp in gets[:H]:
      cp.wait()
    puts_a = [pltpu.async_copy(rowbuf.at[j], acc.at[dstbuf.at[j]], sem_sc,
                               add=True)
              for j in range(H)]
    for cp in gets[H:]:
      cp.wait()
    puts_b = [pltpu.async_copy(rowbuf.at[j], acc.at[dstbuf.at[j]], sem_sc,
                               add=True)
              for j in range(H, KS5)]
    for cp in puts_a + puts_b:
      cp.wait()
    return carry
  lax.fori_loop(0, NST5, stage, 0)

  plsc.subcore_barrier()
  for i in range(CPT // ZR):
    pltpu.sync_copy(acc.at[pl.ds(sid * CPT + i * ZR, ZR)], obuf)
    pltpu.sync_copy(obuf, out_hbm.at[cid, pl.ds(sid * CPT + i * ZR, ZR)])


def _k2_body(degs_ref, x_ref, dinv_ref, u_ref):
  deg = jnp.maximum(degs_ref[0] + degs_ref[1], 1.0)
  dinv = lax.rsqrt(deg)
  dinv_ref[...] = dinv
  u_ref[...] = dinv * x_ref[...]


def _k2_call(degs2, x2):
  return pl.pallas_call(
      _k2_body,
      out_shape=(jax.ShapeDtypeStruct((392, 128), _f32),
                 jax.ShapeDtypeStruct((392, 128), _f32)),
  )(degs2, x2)


BROWS = 3136
NBLK = NPAD // BROWS


def _k4_body(tp0_ref, tp1_ref, dinv_ref, w1_ref, b1_ref, w2_ref,
             gq_ref):
  dinv = dinv_ref[...]
  t = dinv * (tp0_ref[...] + tp1_ref[...])
  h1 = jnp.maximum(t * w1_ref[...] + b1_ref[...], 0.0)
  hm = jnp.dot(h1, w2_ref[...], preferred_element_type=_f32)
  g = dinv * hm
  for q in range(4):
    gq_ref[q] = g[:, q * FQ:(q + 1) * FQ]


def _k4_call(tp0, tp1, dinv, w1, b1, w2):
  col = pl.BlockSpec((BROWS, 1), lambda i: (i, 0))
  full = lambda r, c: pl.BlockSpec((r, c), lambda i: (0, 0))
  return pl.pallas_call(
      _k4_body,
      grid=(NBLK,),
      in_specs=[col, col, col, full(1, HID), full(1, HID),
                full(HID, HID)],
      out_specs=pl.BlockSpec((4, BROWS, FQ), lambda i: (0, i, 0)),
      out_shape=jax.ShapeDtypeStruct((4, NPAD, FQ), _f32),
  )(tp0, tp1, dinv, w1, b1, w2)


B6 = 3136
NB6 = NPAD // B6


def _k6_body(a01_ref, a23_ref, dinv_ref, bat_ref, b2_ref,
             wl1_ref, bl1_ref, wl2_ref, bl2_ref, out_ref, pacc, cacc):
  i = pl.program_id(0)

  @pl.when(i == 0)
  def _():
    pacc[...] = jnp.zeros_like(pacc)
    cacc[...] = jnp.zeros_like(cacc)

  ag = jnp.concatenate(
      [a01_ref[0], a01_ref[1], a23_ref[0], a23_ref[1]], axis=1)
  h2 = jnp.maximum(dinv_ref[...] * ag + b2_ref[...], 0.0)
  gids = lax.broadcasted_iota(jnp.int32, (B6, G), 1)
  ind = (bat_ref[...] == gids).astype(_f32)
  dn = (((0,), (0,)), ((), ()))
  pacc[...] += lax.dot_general(ind, h2, dn, preferred_element_type=_f32)
  ones = jnp.ones((B6, 1), _f32)
  cacc[...] += lax.dot_general(ind, ones, dn, preferred_element_type=_f32)

  @pl.when(i == NB6 - 1)
  def _():
    pooled = pacc[...] / jnp.maximum(cacc[...], 1.0)
    hp = jnp.maximum(
        jnp.dot(pooled, wl1_ref[...], preferred_element_type=_f32)
        + bl1_ref[...], 0.0)
    out_ref[...] = (jnp.dot(hp, wl2_ref[...], preferred_element_type=_f32)
                    + bl2_ref[...])


def _k6_call(a01, a23, dinv, bat, b2, wl1, bl1, wl2p, bl2p):
  blk2 = pl.BlockSpec((2, B6, FQ), lambda i: (0, i, 0))
  blk = lambda c: pl.BlockSpec((B6, c), lambda i: (i, 0))
  full = lambda r, c: pl.BlockSpec((r, c), lambda i: (0, 0))
  return pl.pallas_call(
      _k6_body,
      grid=(NB6,),
      in_specs=[blk2, blk2, blk(1), blk(1),
                full(1, HID), full(HID, HID), full(1, HID),
                full(HID, 128), full(1, 128)],
      out_specs=full(G, 128),
      out_shape=jax.ShapeDtypeStruct((G, 128), _f32),
      scratch_shapes=[pltpu.VMEM((G, HID), _f32), pltpu.VMEM((G, 1), _f32)],
  )(a01, a23, dinv, bat, b2, wl1, bl1, wl2p, bl2p)


def kernel(x, edge_index, batch, W1, b1, W2, b2, Wl1, bl1, Wl2, bl2):
  src = edge_index[0].astype(jnp.int32)
  dst = edge_index[1].astype(jnp.int32)
  bat = batch.astype(jnp.int32)

  loop = jnp.arange(N, dtype=jnp.int32)
  npad_e = EPAD - E - N
  ar = jnp.arange(npad_e, dtype=jnp.int32)
  src_p = jnp.concatenate([src, loop, ar % N]).reshape(ROWS, 128)
  dst_p = jnp.concatenate(
      [dst, loop, N + (ar % (NPAD - N))]).reshape(ROWS, 128)

  x_p = jnp.concatenate([x[:, 0], jnp.zeros((NPAD - N,), _f32)])
  bat_p = jnp.concatenate(
      [bat, jnp.full((NPAD - N,), G, jnp.int32)]).reshape(NPAD, 1)

  degs = _k1_deg(dst_p)
  dinv2, u2 = _k2_call(degs.reshape(2, 392, 128), x_p.reshape(392, 128))
  dinv_f = dinv2.reshape(NPAD)
  u_f = u2.reshape(NPAD)

  tps = _k3_tpre(src_p, dst_p, u_f)

  dinv_c = dinv_f.reshape(NPAD, 1)
  gq = _k4_call(tps[:NPAD].reshape(NPAD, 1), tps[NPAD:].reshape(NPAD, 1),
                dinv_c, W1, b1.reshape(1, HID), W2)

  accs01 = _k5_acc(src_p, dst_p, gq[:2].reshape(2 * NPAD, FQ))
  accs23 = _k5_acc(src_p, dst_p, gq[2:].reshape(2 * NPAD, FQ))

  wl2p = jnp.concatenate([Wl2, jnp.zeros((HID, 128 - OUT), _f32)], axis=1)
  bl2p = jnp.concatenate([bl2, jnp.zeros((128 - OUT,), _f32)]).reshape(1, 128)
  outp = _k6_call(accs01, accs23, dinv_c, bat_p,
                  b2.reshape(1, HID), Wl1, bl1.reshape(1, HID), wl2p, bl2p)
  return outp[:, :OUT]
